# baseline (speedup 1.0000x reference)
import functools

import jax
import jax.numpy as jnp
from jax import lax
from jax.experimental import pallas as pl
from jax.experimental.pallas import tpu as pltpu
from jax.experimental.pallas import tpu_sc as plsc

EMB = 32
B = 4096
L = 200
LA = 128
LB = L - LA
TOTAL = B * L
NC = 2
NS = 16
NW = NC * NS
BAT_W = B // NW
PER_W = TOTAL // NW

_mesh = plsc.VectorSubcoreMesh(core_axis_name="c", subcore_axis_name="s")


@functools.partial(
    pl.kernel,
    out_type=jax.ShapeDtypeStruct((TOTAL, 128), jnp.float32),
    mesh=_mesh,
    compiler_params=pltpu.CompilerParams(use_tc_tiling_on_sc=False),
    scratch_types=[
        pltpu.VMEM((BAT_W, LA), jnp.int32),
        pltpu.VMEM((BAT_W, LB), jnp.int32),
        pltpu.VMEM((2, L, EMB), jnp.float32),
        pltpu.SemaphoreType.DMA,
        pltpu.SemaphoreType.DMA,
    ],
)
def _emb_lookup(xa_hbm, xb_hbm, table_hbm, out_hbm, xav, xbv, rows_v, gsem, wsem):
    wid = lax.axis_index("s") * NC + lax.axis_index("c")
    base = wid * PER_W
    pltpu.sync_copy(xa_hbm.at[pl.ds(wid * BAT_W, BAT_W)], xav)
    pltpu.sync_copy(xb_hbm.at[pl.ds(wid * BAT_W, BAT_W)], xbv)

    def gather(r, slot):
        pltpu.async_copy(
            table_hbm.at[xav.at[r]], rows_v.at[slot, pl.ds(0, LA)], gsem
        )
        pltpu.async_copy(
            table_hbm.at[xbv.at[r]], rows_v.at[slot, pl.ds(LA, LB)], gsem
        )

    def gather_wait(slot):
        pltpu.make_async_copy(
            table_hbm.at[pl.ds(0, LA)], rows_v.at[slot, pl.ds(0, LA)], gsem
        ).wait()
        pltpu.make_async_copy(
            table_hbm.at[pl.ds(0, LB)], rows_v.at[slot, pl.ds(LA, LB)], gsem
        ).wait()

    def write(r, slot):
        pltpu.async_copy(
            rows_v.at[slot],
            out_hbm.at[pl.ds(base + r * L, L), pl.ds(0, EMB)],
            wsem,
        )

    def write_wait(r, slot):
        pltpu.make_async_copy(
            rows_v.at[slot],
            out_hbm.at[pl.ds(base + r * L, L), pl.ds(0, EMB)],
            wsem,
        ).wait()

    gather(0, 0)

    def body(r, _):
        slot = lax.rem(r, 2)
        nslot = 1 - slot

        @pl.when(r >= 1)
        def _():
            write_wait(r - 1, nslot)

        @pl.when(r + 1 < BAT_W)
        def _():
            gather(r + 1, nslot)

        gather_wait(slot)
        write(r, slot)
        return 0

    lax.fori_loop(0, BAT_W, body, 0)
    write_wait(BAT_W - 1, (BAT_W - 1) % 2)


def kernel(x, table):
    xi = x.astype(jnp.int32)
    out = _emb_lookup(xi[:, :LA], xi[:, LA:], table)
    return out[:, :EMB].reshape(B, L, EMB)
